# baseline (speedup 1.0000x reference)
import jax
import jax.numpy as jnp
from jax import lax
from jax.experimental import pallas as pl
from jax.experimental.pallas import tpu as pltpu, tpu_sc as plsc


_NC, _NS, _L = 2, 16, 16
_NW = _NC * _NS
_NR = 8


def _sc_body(x_hbm, pos_hbm, out_hbm, pbuf, xbuf):
    B, S, D = 4, 2048, 4096
    ch = _NR * D
    c = lax.axis_index("c")
    s = lax.axis_index("s")
    wid = s * _NC + c
    rows_per_w = S // _NW
    chunks = rows_per_w // _NR

    def j_loop(j, _):
        srow = wid * rows_per_w + j * _NR
        pltpu.sync_copy(pos_hbm.at[pl.ds(srow * D, ch)], pbuf)

        def b_loop(b, _):
            off = (b * S + srow) * D
            pltpu.sync_copy(x_hbm.at[pl.ds(off, ch)], xbuf)

            def v_loop(i, _):
                o = i * _L
                plsc.addupdate(xbuf.at[pl.ds(o, _L)], pbuf[pl.ds(o, _L)])
                return 0

            lax.fori_loop(0, ch // _L, v_loop, 0, unroll=8)
            pltpu.sync_copy(xbuf, out_hbm.at[pl.ds(off, ch)])
            return 0

        lax.fori_loop(0, B, b_loop, 0)
        return 0

    lax.fori_loop(0, chunks, j_loop, 0)


def kernel(x, pos_table):
    B, S, D = x.shape
    ch = _NR * D
    mesh = plsc.VectorSubcoreMesh(core_axis_name="c", subcore_axis_name="s")
    out_flat = pl.kernel(
        _sc_body,
        out_type=jax.ShapeDtypeStruct((B * S * D,), jnp.float32),
        mesh=mesh,
        scratch_types=[
            pltpu.VMEM((ch,), jnp.float32),
            pltpu.VMEM((ch,), jnp.float32),
        ],
    )(x.reshape(-1), pos_table.reshape(-1))
    return out_flat.reshape(B, S, D)


# speedup vs baseline: 1.1797x; 1.1797x over previous
import jax
import jax.numpy as jnp
from jax import lax
from jax.experimental import pallas as pl
from jax.experimental.pallas import tpu as pltpu, tpu_sc as plsc


_NC, _NS, _L = 2, 16, 16
_NW = _NC * _NS
_NR = 8


def _sc_body(x_hbm, pos_hbm, out_hbm, pbuf, xb0, xb1,
             sin0, sin1, sout0, sout1):
    B, S, D = 4, 2048, 4096
    ch = _NR * D
    c = lax.axis_index("c")
    s = lax.axis_index("s")
    wid = s * _NC + c
    rows_per_w = S // _NW
    base_row = wid * rows_per_w
    nsteps = (rows_per_w // _NR) * B

    def x_off(k):
        j = k // B
        b = k - j * B
        return (b * S + base_row + j * _NR) * D

    def add_loop(xb):
        @plsc.parallel_loop(0, ch, _L, unroll=8)
        def _(o):
            plsc.addupdate(xb.at[pl.ds(o, _L)], pbuf[pl.ds(o, _L)])

    pltpu.async_copy(x_hbm.at[pl.ds(x_off(0), ch)], xb0, sin0)

    def g_loop(g, _):
        k0 = 2 * g
        k1 = k0 + 1
        j0 = k0 // B

        @pl.when(lax.rem(k0, B) == 0)
        def _():
            pltpu.sync_copy(pos_hbm.at[pl.ds((base_row + j0 * _NR) * D, ch)],
                            pbuf)

        @pl.when(g > 0)
        def _():
            pltpu.make_async_copy(xb1, out_hbm.at[pl.ds(x_off(k1 - 2), ch)],
                                  sout1).wait()
        pltpu.async_copy(x_hbm.at[pl.ds(x_off(k1), ch)], xb1, sin1)

        pltpu.make_async_copy(x_hbm.at[pl.ds(x_off(k0), ch)], xb0, sin0).wait()
        add_loop(xb0)
        pltpu.async_copy(xb0, out_hbm.at[pl.ds(x_off(k0), ch)], sout0)

        @pl.when(k0 + 2 < nsteps)
        def _():
            pltpu.make_async_copy(xb0, out_hbm.at[pl.ds(x_off(k0), ch)],
                                  sout0).wait()
            pltpu.async_copy(x_hbm.at[pl.ds(x_off(k0 + 2), ch)], xb0, sin0)

        pltpu.make_async_copy(x_hbm.at[pl.ds(x_off(k1), ch)], xb1, sin1).wait()
        add_loop(xb1)
        pltpu.async_copy(xb1, out_hbm.at[pl.ds(x_off(k1), ch)], sout1)
        return 0

    lax.fori_loop(0, nsteps // 2, g_loop, 0)

    pltpu.make_async_copy(xb0, out_hbm.at[pl.ds(x_off(nsteps - 2), ch)],
                          sout0).wait()
    pltpu.make_async_copy(xb1, out_hbm.at[pl.ds(x_off(nsteps - 1), ch)],
                          sout1).wait()


def kernel(x, pos_table):
    B, S, D = x.shape
    ch = _NR * D
    mesh = plsc.VectorSubcoreMesh(core_axis_name="c", subcore_axis_name="s")
    out_flat = pl.kernel(
        _sc_body,
        out_type=jax.ShapeDtypeStruct((B * S * D,), jnp.float32),
        mesh=mesh,
        scratch_types=[
            pltpu.VMEM((ch,), jnp.float32),
            pltpu.VMEM((ch,), jnp.float32),
            pltpu.VMEM((ch,), jnp.float32),
            pltpu.SemaphoreType.DMA,
            pltpu.SemaphoreType.DMA,
            pltpu.SemaphoreType.DMA,
            pltpu.SemaphoreType.DMA,
        ],
    )(x.reshape(-1), pos_table.reshape(-1))
    return out_flat.reshape(B, S, D)


# speedup vs baseline: 1.2563x; 1.0649x over previous
import jax
import jax.numpy as jnp
from jax import lax
from jax.experimental import pallas as pl
from jax.experimental.pallas import tpu as pltpu, tpu_sc as plsc


_NC, _NS, _L = 2, 16, 16
_NW = _NC * _NS
_NR = 4


def _sc_body(x_hbm, pos_hbm, out_hbm,
             pb0, pb1, xb0, xb1, xb2, xb3,
             sp0, sp1, si0, si1, si2, si3, so0, so1, so2, so3):
    B, S, D = 4, 2048, 4096
    ch = _NR * D
    c = lax.axis_index("c")
    s = lax.axis_index("s")
    wid = s * _NC + c
    rows_per_w = S // _NW
    base_row = wid * rows_per_w
    nj = rows_per_w // _NR
    nsteps = nj * B

    xbs = [xb0, xb1, xb2, xb3]
    sis = [si0, si1, si2, si3]
    sos = [so0, so1, so2, so3]
    pbs = [pb0, pb1]
    sps = [sp0, sp1]

    def x_off(k):
        j = k // B
        b = k - j * B
        return (b * S + base_row + j * _NR) * D

    def p_off(j):
        return (base_row + j * _NR) * D

    def add_loop(xb, pb):
        @plsc.parallel_loop(0, ch, _L, unroll=8)
        def _(o):
            plsc.addupdate(xb.at[pl.ds(o, _L)], pb[pl.ds(o, _L)])

    pltpu.async_copy(pos_hbm.at[pl.ds(p_off(0), ch)], pb0, sp0)
    pltpu.async_copy(x_hbm.at[pl.ds(x_off(0), ch)], xb0, si0)
    pltpu.async_copy(x_hbm.at[pl.ds(x_off(1), ch)], xb1, si1)

    def gg_loop(gg, _):
        for dj in range(2):
            g = 2 * gg + dj
            pltpu.make_async_copy(pos_hbm.at[pl.ds(p_off(g), ch)],
                                  pbs[dj], sps[dj]).wait()
            @pl.when(g + 1 < nj)
            def _():
                pltpu.async_copy(pos_hbm.at[pl.ds(p_off(g + 1), ch)],
                                 pbs[1 - dj], sps[1 - dj])
            for d in range(4):
                k = 4 * g + d
                dn = (d + 2) % 4
                @pl.when(k >= 2)
                def _():
                    pltpu.make_async_copy(
                        xbs[dn], out_hbm.at[pl.ds(x_off(k - 2), ch)],
                        sos[dn]).wait()
                @pl.when(k + 2 < nsteps)
                def _():
                    pltpu.async_copy(x_hbm.at[pl.ds(x_off(k + 2), ch)],
                                     xbs[dn], sis[dn])
                pltpu.make_async_copy(x_hbm.at[pl.ds(x_off(k), ch)],
                                      xbs[d], sis[d]).wait()
                add_loop(xbs[d], pbs[dj])
                pltpu.async_copy(xbs[d], out_hbm.at[pl.ds(x_off(k), ch)],
                                 sos[d])
        return 0

    lax.fori_loop(0, nj // 2, gg_loop, 0)

    pltpu.make_async_copy(xb2, out_hbm.at[pl.ds(x_off(nsteps - 2), ch)],
                          so2).wait()
    pltpu.make_async_copy(xb3, out_hbm.at[pl.ds(x_off(nsteps - 1), ch)],
                          so3).wait()


def kernel(x, pos_table):
    B, S, D = x.shape
    ch = _NR * D
    mesh = plsc.VectorSubcoreMesh(core_axis_name="c", subcore_axis_name="s")
    out_flat = pl.kernel(
        _sc_body,
        out_type=jax.ShapeDtypeStruct((B * S * D,), jnp.float32),
        mesh=mesh,
        scratch_types=(
            [pltpu.VMEM((ch,), jnp.float32) for _ in range(6)]
            + [pltpu.SemaphoreType.DMA for _ in range(10)]
        ),
    )(x.reshape(-1), pos_table.reshape(-1))
    return out_flat.reshape(B, S, D)
